# baseline (speedup 1.0000x reference)
import functools

import jax
import jax.numpy as jnp
from jax import lax
from jax.experimental import pallas as pl
from jax.experimental.pallas import tpu as pltpu
from jax.experimental.pallas import tpu_sc as plsc

N = 10000
E = 320000
D_IN = 128
D_EDGE = 16
H = 128
L = 5

NC = 2
NS = 16
NW = NC * NS
E2 = E // 2
EPW = E2 // NW
GC = 40
NCHUNK = EPW // GC
ACH = GC
NACH = N // ACH
F32 = jnp.float32


@functools.lru_cache(maxsize=None)
def _sc_mesh():
    return plsc.VectorSubcoreMesh(
        core_axis_name="c", subcore_axis_name="s",
        num_cores=NC, num_subcores=NS)


@functools.lru_cache(maxsize=None)
def _build_sc_gather():
    @functools.partial(
        pl.kernel,
        out_type=[jax.ShapeDtypeStruct((E2, H), F32),
                  jax.ShapeDtypeStruct((E2, H), F32)],
        mesh=_sc_mesh(),
        scratch_types=(
            [pltpu.VMEM((NCHUNK, GC), jnp.int32)] * 2
            + [pltpu.VMEM((GC, H), F32)] * 8
            + [pltpu.SemaphoreType.DMA] * 16
        ),
    )
    def gather_k(ps_hbm, pr_hbm, s3_hbm, r3_hbm, gs_hbm, gr_hbm,
                 idx_s, idx_r, bs0, bs1, bs2, bs3, br0, br1, br2, br3,
                 gsa, gsb, gsc, gsd, gra, grb, grc, grd,
                 wsa, wsb, wsc, wsd, wra, wrb, wrc, wrd):
        wid = lax.axis_index("c") * NS + lax.axis_index("s")
        base = wid * EPW
        bs = (bs0, bs1, bs2, bs3)
        br = (br0, br1, br2, br3)
        gsm_s = (gsa, gsb, gsc, gsd)
        gsm_r = (gra, grb, grc, grd)
        wsm_s = (wsa, wsb, wsc, wsd)
        wsm_r = (wra, wrb, wrc, wrd)

        pltpu.sync_copy(s3_hbm.at[wid], idx_s)
        pltpu.sync_copy(r3_hbm.at[wid], idx_r)

        def fire_g(g, sl):
            pltpu.async_copy(ps_hbm.at[idx_s.at[g]], bs[sl], gsm_s[sl])
            pltpu.async_copy(pr_hbm.at[idx_r.at[g]], br[sl], gsm_r[sl])

        def drain_g(sl):
            pltpu.make_async_copy(gs_hbm.at[pl.ds(0, GC)], bs[sl],
                                  gsm_s[sl]).wait()
            pltpu.make_async_copy(gs_hbm.at[pl.ds(0, GC)], br[sl],
                                  gsm_r[sl]).wait()

        def fire_w(g, sl):
            off = base + g * GC
            pltpu.async_copy(bs[sl], gs_hbm.at[pl.ds(off, GC)], wsm_s[sl])
            pltpu.async_copy(br[sl], gr_hbm.at[pl.ds(off, GC)], wsm_r[sl])

        def drain_w(sl):
            pltpu.make_async_copy(bs[sl], gs_hbm.at[pl.ds(0, GC)],
                                  wsm_s[sl]).wait()
            pltpu.make_async_copy(br[sl], gr_hbm.at[pl.ds(0, GC)],
                                  wsm_r[sl]).wait()

        def step(g, sl, fire_next):
            nsl = (sl + 3) % 4

            if isinstance(g, int):
                if g >= 1:
                    drain_w(nsl)
            else:
                @pl.when(g >= 1)
                def _():
                    drain_w(nsl)

            if fire_next:
                fire_g(g + 3, nsl)
            drain_g(sl)
            fire_w(g, sl)

        fire_g(0, 0)
        fire_g(1, 1)
        fire_g(2, 2)

        def body(go, carry):
            for b in range(4):
                step(4 * go + b, b, True)
            return carry

        lax.fori_loop(0, (NCHUNK - 5) // 4, body, 0)
        for g in range(NCHUNK - 5, NCHUNK):
            step(g, g % 4, g + 3 < NCHUNK)
        drain_w((NCHUNK - 1) % 4)

    return gather_k


def _sc_gather(ps, pr, s3, r3):
    return _build_sc_gather()(ps, pr, s3, r3)


@functools.lru_cache(maxsize=None)
def _build_sc_scatter():
    @functools.partial(
        pl.kernel,
        out_type=jax.ShapeDtypeStruct((NC, N, H), F32),
        mesh=_sc_mesh(),
        scratch_types=(
            [pltpu.VMEM((NCHUNK, GC), jnp.int32)]
            + [pltpu.VMEM((GC, H), F32)] * 4
            + [pltpu.VMEM_SHARED((N, H), F32)]
            + [pltpu.SemaphoreType.DMA] * 8
        ),
    )
    def scatter_k(e_hbm, r3_hbm, out_hbm, idx_v, eb0, eb1, eb2, eb3, agg_sh,
                  la, lb, lc, ld, sa, sb, sc, sd):
        obuf = eb0
        cid = lax.axis_index("c")
        sid = lax.axis_index("s")
        wid = cid * NS + sid
        base = wid * EPW
        eb = (eb0, eb1, eb2, eb3)
        lsm = (la, lb, lc, ld)
        ssm = (sa, sb, sc, sd)

        pltpu.sync_copy(r3_hbm.at[wid], idx_v)

        zero16 = jnp.zeros((16,), F32)

        def zv(t, carry):
            obuf[t // 8, pl.ds((t % 8) * 16, 16)] = zero16
            return carry

        lax.fori_loop(0, ACH * 8, zv, 0)

        nk = (NACH + NS - 1) // NS

        def zs(k, carry):
            ch = sid + k * NS

            @pl.when(ch < NACH)
            def _():
                pltpu.sync_copy(obuf, agg_sh.at[pl.ds(ch * ACH, ACH)])

            return carry

        lax.fori_loop(0, nk, zs, 0)
        plsc.subcore_barrier()

        def fire_l(g, sl):
            pltpu.async_copy(e_hbm.at[pl.ds(base + g * GC, GC)], eb[sl],
                             lsm[sl])

        def drain_l(sl):
            pltpu.make_async_copy(e_hbm.at[pl.ds(0, GC)], eb[sl],
                                  lsm[sl]).wait()

        def fire_s(g, sl):
            pltpu.async_copy(eb[sl], agg_sh.at[idx_v.at[g]], ssm[sl],
                             add=True)

        def drain_s(sl):
            pltpu.make_async_copy(eb[sl], agg_sh.at[pl.ds(0, GC)],
                                  ssm[sl]).wait()

        def step(g, sl, fire_next):
            nsl = (sl + 3) % 4

            if isinstance(g, int):
                if g >= 1:
                    drain_s(nsl)
            else:
                @pl.when(g >= 1)
                def _():
                    drain_s(nsl)

            if fire_next:
                fire_l(g + 3, nsl)
            drain_l(sl)
            fire_s(g, sl)

        fire_l(0, 0)
        fire_l(1, 1)
        fire_l(2, 2)

        def body(go, carry):
            for b in range(4):
                step(4 * go + b, b, True)
            return carry

        lax.fori_loop(0, (NCHUNK - 5) // 4, body, 0)
        for g in range(NCHUNK - 5, NCHUNK):
            step(g, g % 4, g + 3 < NCHUNK)
        drain_s((NCHUNK - 1) % 4)
        plsc.subcore_barrier()

        def rd(k, carry):
            ch = sid + k * NS

            @pl.when(ch < NACH)
            def _():
                rows = pl.ds(ch * ACH, ACH)
                pltpu.sync_copy(agg_sh.at[rows], obuf)
                pltpu.sync_copy(obuf, out_hbm.at[cid, rows])

            return carry

        lax.fori_loop(0, nk, rd, 0)

    return scatter_k


def _sc_scatter(e, r3):
    return _build_sc_scatter()(e, r3)


BE = 2000
BN = 2000


def _full(shape):
    return pl.BlockSpec(shape, lambda i: (0,) * len(shape))


def _rows(b, w):
    return pl.BlockSpec((b, w), lambda i: (i, 0))


def _ln_rows(x, s, b):
    m = jnp.mean(x, axis=-1, keepdims=True)
    v = jnp.mean((x - m) ** 2, axis=-1, keepdims=True)
    return (x - m) * lax.rsqrt(v + 1e-6) * s + b


def _dot(a, b):
    return jnp.dot(a, b, preferred_element_type=F32)


def _edge_encoder(edges, w1, b1, w2, b2, blk_off):
    def body(x_ref, w1_ref, b1_ref, w2_ref, b2_ref, o_ref):
        h = jnp.maximum(_dot(x_ref[...], w1_ref[...]) + b1_ref[...], 0.0)
        o_ref[...] = _dot(h, w2_ref[...]) + b2_ref[...]

    return pl.pallas_call(
        body,
        grid=(E2 // BE,),
        in_specs=[pl.BlockSpec((BE, D_EDGE), lambda i: (i + blk_off, 0)),
                  _full((D_EDGE, 2 * H)), _full((1, 2 * H)),
                  _full((2 * H, H)), _full((1, H))],
        out_specs=_rows(BE, H),
        out_shape=jax.ShapeDtypeStruct((E2, H), F32),
    )(edges, w1, b1.reshape(1, -1), w2, b2.reshape(1, -1))


def _node_encoder(nodes, w1, b1, w2, b2, wps, wpr):
    def body(x_ref, w1_ref, b1_ref, w2_ref, b2_ref, wps_ref, wpr_ref,
             n_ref, ps_ref, pr_ref):
        h = jnp.maximum(_dot(x_ref[...], w1_ref[...]) + b1_ref[...], 0.0)
        n = _dot(h, w2_ref[...]) + b2_ref[...]
        n_ref[...] = n
        ps_ref[...] = _dot(n, wps_ref[...])
        pr_ref[...] = _dot(n, wpr_ref[...])

    return pl.pallas_call(
        body,
        grid=(N // BN,),
        in_specs=[_rows(BN, D_IN), _full((D_IN, 2 * H)), _full((1, 2 * H)),
                  _full((2 * H, H)), _full((1, H)), _full((H, H)), _full((H, H))],
        out_specs=[_rows(BN, H)] * 3,
        out_shape=[jax.ShapeDtypeStruct((N, H), F32)] * 3,
    )(nodes, w1, b1.reshape(1, -1), w2, b2.reshape(1, -1), wps, wpr)


def _edge_layer(e, gs, gr, w1e, w2, b1, b2, ln_s, ln_b):
    apply_ln = ln_s is not None

    def body(*refs):
        if apply_ln:
            (e_ref, gs_ref, gr_ref, w1_ref, w2_ref, b1_ref, b2_ref,
             s_ref, lb_ref, o_ref) = refs
            x = _ln_rows(e_ref[...], s_ref[...], lb_ref[...])
        else:
            (e_ref, gs_ref, gr_ref, w1_ref, w2_ref, b1_ref, b2_ref,
             o_ref) = refs
            x = e_ref[...]
        h = jnp.maximum(
            _dot(x, w1_ref[...]) + gs_ref[...] + gr_ref[...] + b1_ref[...],
            0.0)
        o_ref[...] = x + _dot(h, w2_ref[...]) + b2_ref[...]

    in_specs = [_rows(BE, H), _rows(BE, H), _rows(BE, H),
                _full((H, H)), _full((H, H)), _full((1, H)), _full((1, H))]
    args = [e, gs, gr, w1e, w2, b1.reshape(1, -1), b2.reshape(1, -1)]
    if apply_ln:
        in_specs += [_full((1, H)), _full((1, H))]
        args += [ln_s.reshape(1, -1), ln_b.reshape(1, -1)]

    return pl.pallas_call(
        body,
        grid=(E2 // BE,),
        in_specs=in_specs,
        out_specs=_rows(BE, H),
        out_shape=jax.ShapeDtypeStruct((E2, H), F32),
    )(*args)


def _node_layer_mid(n, aggp, aggq, w1n, w1a, w2, b1, b2, ln_s, ln_b, wps, wpr):
    def body(n_ref, ap_ref, bp_ref, w1n_ref, w1a_ref, w2_ref, b1_ref, b2_ref,
             s_ref, lb_ref, wps_ref, wpr_ref, n2_ref, ps_ref, pr_ref):
        nn = n_ref[...]
        agg = (ap_ref[0] + ap_ref[1]) + (bp_ref[0] + bp_ref[1])
        h = jnp.maximum(
            _dot(nn, w1n_ref[...]) + _dot(agg, w1a_ref[...]) + b1_ref[...], 0.0)
        n2 = nn + _dot(h, w2_ref[...]) + b2_ref[...]
        n2 = _ln_rows(n2, s_ref[...], lb_ref[...])
        n2_ref[...] = n2
        ps_ref[...] = _dot(n2, wps_ref[...])
        pr_ref[...] = _dot(n2, wpr_ref[...])

    return pl.pallas_call(
        body,
        grid=(N // BN,),
        in_specs=[_rows(BN, H),
                  pl.BlockSpec((NC, BN, H), lambda i: (0, i, 0)),
                  pl.BlockSpec((NC, BN, H), lambda i: (0, i, 0)),
                  _full((H, H)), _full((H, H)), _full((H, H)),
                  _full((1, H)), _full((1, H)), _full((1, H)), _full((1, H)),
                  _full((H, H)), _full((H, H))],
        out_specs=[_rows(BN, H)] * 3,
        out_shape=[jax.ShapeDtypeStruct((N, H), F32)] * 3,
    )(n, aggp, aggq, w1n, w1a, w2, b1.reshape(1, -1), b2.reshape(1, -1),
      ln_s.reshape(1, -1), ln_b.reshape(1, -1), wps, wpr)


def _node_layer_last(n, aggp, aggq, w1n, w1a, w2, b1, b2, dw1, db1, dw2, db2):
    def body(n_ref, ap_ref, bp_ref, w1n_ref, w1a_ref, w2_ref, b1_ref, b2_ref,
             dw1_ref, db1_ref, dw2_ref, db2_ref, o_ref):
        nn = n_ref[...]
        agg = (ap_ref[0] + ap_ref[1]) + (bp_ref[0] + bp_ref[1])
        h = jnp.maximum(
            _dot(nn, w1n_ref[...]) + _dot(agg, w1a_ref[...]) + b1_ref[...], 0.0)
        n2 = nn + _dot(h, w2_ref[...]) + b2_ref[...]
        hd = jnp.maximum(_dot(n2, dw1_ref[...]) + db1_ref[...], 0.0)
        o_ref[...] = _dot(hd, dw2_ref[...]) + db2_ref[...]

    return pl.pallas_call(
        body,
        grid=(N // BN,),
        in_specs=[_rows(BN, H),
                  pl.BlockSpec((NC, BN, H), lambda i: (0, i, 0)),
                  pl.BlockSpec((NC, BN, H), lambda i: (0, i, 0)),
                  _full((H, H)), _full((H, H)), _full((H, H)),
                  _full((1, H)), _full((1, H)),
                  _full((H, H)), _full((1, H)), _full((H, 2 * H)), _full((1, 2 * H))],
        out_specs=_rows(BN, 2 * H),
        out_shape=jax.ShapeDtypeStruct((N, 2 * H), F32),
    )(n, aggp, aggq, w1n, w1a, w2, b1.reshape(1, -1), b2.reshape(1, -1),
      dw1, db1.reshape(1, -1), dw2, db2.reshape(1, -1))


def kernel(nodes, edges, senders, receivers, params):
    p = params
    pe_w1 = p['pe_W1']
    pn_w1 = p['pn_W1']

    sA3 = senders[:E2].reshape(NW, NCHUNK, GC)
    sB3 = senders[E2:].reshape(NW, NCHUNK, GC)
    rA3 = receivers[:E2].reshape(NW, NCHUNK, GC)
    rB3 = receivers[E2:].reshape(NW, NCHUNK, GC)

    eA = _edge_encoder(edges, p['enc_e_W1'], p['enc_e_b1'],
                       p['enc_e_W2'], p['enc_e_b2'], 0)
    eB = _edge_encoder(edges, p['enc_e_W1'], p['enc_e_b1'],
                       p['enc_e_W2'], p['enc_e_b2'], E2 // BE)
    n, ps, pr = _node_encoder(nodes, p['enc_n_W1'], p['enc_n_b1'],
                              p['enc_n_W2'], p['enc_n_b2'],
                              pe_w1[0, H:2 * H], pe_w1[0, 2 * H:])

    out = None
    for i in range(L):
        lnp = (p['ln_e_s'][i - 1] if i > 0 else None,
               p['ln_e_b'][i - 1] if i > 0 else None)
        gsA, grA = _sc_gather(ps, pr, sA3, rA3)
        eA = _edge_layer(eA, gsA, grA, pe_w1[i, :H], p['pe_W2'][i],
                         p['pe_b1'][i], p['pe_b2'][i], *lnp)
        gsB, grB = _sc_gather(ps, pr, sB3, rB3)
        aggpA = _sc_scatter(eA, rA3)
        eB = _edge_layer(eB, gsB, grB, pe_w1[i, :H], p['pe_W2'][i],
                         p['pe_b1'][i], p['pe_b2'][i], *lnp)
        aggpB = _sc_scatter(eB, rB3)
        if i < L - 1:
            n, ps, pr = _node_layer_mid(
                n, aggpA, aggpB, pn_w1[i, :H], pn_w1[i, H:], p['pn_W2'][i],
                p['pn_b1'][i], p['pn_b2'][i],
                p['ln_n_s'][i], p['ln_n_b'][i],
                pe_w1[i + 1, H:2 * H], pe_w1[i + 1, 2 * H:])
        else:
            out = _node_layer_last(
                n, aggpA, aggpB, pn_w1[i, :H], pn_w1[i, H:], p['pn_W2'][i],
                p['pn_b1'][i], p['pn_b2'][i],
                p['dec_W1'], p['dec_b1'], p['dec_W2'], p['dec_b2'])
    return out


# speedup vs baseline: 1.0034x; 1.0034x over previous
import functools

import jax
import jax.numpy as jnp
from jax import lax
from jax.experimental import pallas as pl
from jax.experimental.pallas import tpu as pltpu
from jax.experimental.pallas import tpu_sc as plsc

N = 10000
E = 320000
D_IN = 128
D_EDGE = 16
H = 128
L = 5

NC = 2
NS = 16
NW = NC * NS
E2 = E // 2
EPW = E2 // NW
GC = 40
NCHUNK = EPW // GC
ACH = 80
NACH = N // ACH
RCH = 200
NRCH = N // RCH
F32 = jnp.float32


@functools.lru_cache(maxsize=None)
def _sc_mesh():
    return plsc.VectorSubcoreMesh(
        core_axis_name="c", subcore_axis_name="s",
        num_cores=NC, num_subcores=NS)


@functools.lru_cache(maxsize=None)
def _build_sc_gather():
    @functools.partial(
        pl.kernel,
        out_type=[jax.ShapeDtypeStruct((E2, H), F32),
                  jax.ShapeDtypeStruct((E2, H), F32)],
        mesh=_sc_mesh(),
        scratch_types=(
            [pltpu.VMEM((NCHUNK, GC), jnp.int32)] * 2
            + [pltpu.VMEM((GC, H), F32)] * 8
            + [pltpu.SemaphoreType.DMA] * 16
        ),
    )
    def gather_k(ps_hbm, pr_hbm, s3_hbm, r3_hbm, gs_hbm, gr_hbm,
                 idx_s, idx_r, bs0, bs1, bs2, bs3, br0, br1, br2, br3,
                 gsa, gsb, gsc, gsd, gra, grb, grc, grd,
                 wsa, wsb, wsc, wsd, wra, wrb, wrc, wrd):
        wid = lax.axis_index("c") * NS + lax.axis_index("s")
        base = wid * EPW
        bs = (bs0, bs1, bs2, bs3)
        br = (br0, br1, br2, br3)
        gsm_s = (gsa, gsb, gsc, gsd)
        gsm_r = (gra, grb, grc, grd)
        wsm_s = (wsa, wsb, wsc, wsd)
        wsm_r = (wra, wrb, wrc, wrd)

        pltpu.sync_copy(s3_hbm.at[wid], idx_s)
        pltpu.sync_copy(r3_hbm.at[wid], idx_r)

        def fire_g(g, sl):
            pltpu.async_copy(ps_hbm.at[idx_s.at[g]], bs[sl], gsm_s[sl])
            pltpu.async_copy(pr_hbm.at[idx_r.at[g]], br[sl], gsm_r[sl])

        def drain_g(sl):
            pltpu.make_async_copy(gs_hbm.at[pl.ds(0, GC)], bs[sl],
                                  gsm_s[sl]).wait()
            pltpu.make_async_copy(gs_hbm.at[pl.ds(0, GC)], br[sl],
                                  gsm_r[sl]).wait()

        def fire_w(g, sl):
            off = base + g * GC
            pltpu.async_copy(bs[sl], gs_hbm.at[pl.ds(off, GC)], wsm_s[sl])
            pltpu.async_copy(br[sl], gr_hbm.at[pl.ds(off, GC)], wsm_r[sl])

        def drain_w(sl):
            pltpu.make_async_copy(bs[sl], gs_hbm.at[pl.ds(0, GC)],
                                  wsm_s[sl]).wait()
            pltpu.make_async_copy(br[sl], gr_hbm.at[pl.ds(0, GC)],
                                  wsm_r[sl]).wait()

        def step(g, sl, fire_next):
            nsl = (sl + 3) % 4

            if isinstance(g, int):
                if g >= 1:
                    drain_w(nsl)
            else:
                @pl.when(g >= 1)
                def _():
                    drain_w(nsl)

            if fire_next:
                fire_g(g + 3, nsl)
            drain_g(sl)
            fire_w(g, sl)

        fire_g(0, 0)
        fire_g(1, 1)
        fire_g(2, 2)

        def body(go, carry):
            for b in range(4):
                step(4 * go + b, b, True)
            return carry

        lax.fori_loop(0, (NCHUNK - 5) // 4, body, 0)
        for g in range(NCHUNK - 5, NCHUNK):
            step(g, g % 4, g + 3 < NCHUNK)
        drain_w((NCHUNK - 1) % 4)

    return gather_k


def _sc_gather(ps, pr, s3, r3):
    return _build_sc_gather()(ps, pr, s3, r3)


@functools.lru_cache(maxsize=None)
def _build_sc_scatter():
    @functools.partial(
        pl.kernel,
        out_type=jax.ShapeDtypeStruct((NC, N, H), F32),
        mesh=_sc_mesh(),
        scratch_types=(
            [pltpu.VMEM((NCHUNK, GC), jnp.int32)]
            + [pltpu.VMEM((GC, H), F32)] * 4
            + [pltpu.VMEM((ACH, H), F32)]
            + [pltpu.VMEM_SHARED((N, H), F32)]
            + [pltpu.SemaphoreType.DMA] * 8
        ),
    )
    def scatter_k(e_hbm, r3_hbm, out_hbm, idx_v, eb0, eb1, eb2, eb3, obuf,
                  agg_sh, la, lb, lc, ld, sa, sb, sc, sd):
        cid = lax.axis_index("c")
        sid = lax.axis_index("s")
        wid = cid * NS + sid
        base = wid * EPW
        eb = (eb0, eb1, eb2, eb3)
        lsm = (la, lb, lc, ld)
        ssm = (sa, sb, sc, sd)

        pltpu.sync_copy(r3_hbm.at[wid], idx_v)

        zero16 = jnp.zeros((16,), F32)

        def zv(t, carry):
            obuf[t // 8, pl.ds((t % 8) * 16, 16)] = zero16
            return carry

        lax.fori_loop(0, ACH * 8, zv, 0)

        nk = (NACH + NS - 1) // NS

        def zs(k, carry):
            ch = sid + k * NS

            @pl.when(ch < NACH)
            def _():
                pltpu.sync_copy(obuf, agg_sh.at[pl.ds(ch * ACH, ACH)])

            return carry

        lax.fori_loop(0, nk, zs, 0)
        plsc.subcore_barrier()

        def fire_l(g, sl):
            pltpu.async_copy(e_hbm.at[pl.ds(base + g * GC, GC)], eb[sl],
                             lsm[sl])

        def drain_l(sl):
            pltpu.make_async_copy(e_hbm.at[pl.ds(0, GC)], eb[sl],
                                  lsm[sl]).wait()

        def fire_s(g, sl):
            pltpu.async_copy(eb[sl], agg_sh.at[idx_v.at[g]], ssm[sl],
                             add=True)

        def drain_s(sl):
            pltpu.make_async_copy(eb[sl], agg_sh.at[pl.ds(0, GC)],
                                  ssm[sl]).wait()

        def step(g, sl, fire_next):
            nsl = (sl + 3) % 4

            if isinstance(g, int):
                if g >= 1:
                    drain_s(nsl)
            else:
                @pl.when(g >= 1)
                def _():
                    drain_s(nsl)

            if fire_next:
                fire_l(g + 3, nsl)
            drain_l(sl)
            fire_s(g, sl)

        fire_l(0, 0)
        fire_l(1, 1)
        fire_l(2, 2)

        def body(go, carry):
            for b in range(4):
                step(4 * go + b, b, True)
            return carry

        lax.fori_loop(0, (NCHUNK - 5) // 4, body, 0)
        for g in range(NCHUNK - 5, NCHUNK):
            step(g, g % 4, g + 3 < NCHUNK)
        drain_s((NCHUNK - 1) % 4)
        plsc.subcore_barrier()

        nr = (NRCH + NS - 1) // NS

        def rd(k, carry):
            ch = sid + k * NS

            @pl.when(ch < NRCH)
            def _():
                rows = pl.ds(ch * RCH, RCH)
                pltpu.sync_copy(agg_sh.at[rows], out_hbm.at[cid, rows])

            return carry

        lax.fori_loop(0, nr, rd, 0)

    return scatter_k


def _sc_scatter(e, r3):
    return _build_sc_scatter()(e, r3)


BE = 2000
BN = 2000


def _full(shape):
    return pl.BlockSpec(shape, lambda i: (0,) * len(shape))


def _rows(b, w):
    return pl.BlockSpec((b, w), lambda i: (i, 0))


def _ln_rows(x, s, b):
    m = jnp.mean(x, axis=-1, keepdims=True)
    v = jnp.mean((x - m) ** 2, axis=-1, keepdims=True)
    return (x - m) * lax.rsqrt(v + 1e-6) * s + b


def _dot(a, b):
    return jnp.dot(a, b, preferred_element_type=F32)


def _edge_encoder(edges, w1, b1, w2, b2, blk_off):
    def body(x_ref, w1_ref, b1_ref, w2_ref, b2_ref, o_ref):
        h = jnp.maximum(_dot(x_ref[...], w1_ref[...]) + b1_ref[...], 0.0)
        o_ref[...] = _dot(h, w2_ref[...]) + b2_ref[...]

    return pl.pallas_call(
        body,
        grid=(E2 // BE,),
        in_specs=[pl.BlockSpec((BE, D_EDGE), lambda i: (i + blk_off, 0)),
                  _full((D_EDGE, 2 * H)), _full((1, 2 * H)),
                  _full((2 * H, H)), _full((1, H))],
        out_specs=_rows(BE, H),
        out_shape=jax.ShapeDtypeStruct((E2, H), F32),
    )(edges, w1, b1.reshape(1, -1), w2, b2.reshape(1, -1))


def _node_encoder(nodes, w1, b1, w2, b2, wps, wpr):
    def body(x_ref, w1_ref, b1_ref, w2_ref, b2_ref, wps_ref, wpr_ref,
             n_ref, ps_ref, pr_ref):
        h = jnp.maximum(_dot(x_ref[...], w1_ref[...]) + b1_ref[...], 0.0)
        n = _dot(h, w2_ref[...]) + b2_ref[...]
        n_ref[...] = n
        ps_ref[...] = _dot(n, wps_ref[...])
        pr_ref[...] = _dot(n, wpr_ref[...])

    return pl.pallas_call(
        body,
        grid=(N // BN,),
        in_specs=[_rows(BN, D_IN), _full((D_IN, 2 * H)), _full((1, 2 * H)),
                  _full((2 * H, H)), _full((1, H)), _full((H, H)), _full((H, H))],
        out_specs=[_rows(BN, H)] * 3,
        out_shape=[jax.ShapeDtypeStruct((N, H), F32)] * 3,
    )(nodes, w1, b1.reshape(1, -1), w2, b2.reshape(1, -1), wps, wpr)


def _edge_layer(e, gs, gr, w1e, w2, b1, b2, ln_s, ln_b):
    apply_ln = ln_s is not None

    def body(*refs):
        if apply_ln:
            (e_ref, gs_ref, gr_ref, w1_ref, w2_ref, b1_ref, b2_ref,
             s_ref, lb_ref, o_ref) = refs
            x = _ln_rows(e_ref[...], s_ref[...], lb_ref[...])
        else:
            (e_ref, gs_ref, gr_ref, w1_ref, w2_ref, b1_ref, b2_ref,
             o_ref) = refs
            x = e_ref[...]
        h = jnp.maximum(
            _dot(x, w1_ref[...]) + gs_ref[...] + gr_ref[...] + b1_ref[...],
            0.0)
        o_ref[...] = x + _dot(h, w2_ref[...]) + b2_ref[...]

    in_specs = [_rows(BE, H), _rows(BE, H), _rows(BE, H),
                _full((H, H)), _full((H, H)), _full((1, H)), _full((1, H))]
    args = [e, gs, gr, w1e, w2, b1.reshape(1, -1), b2.reshape(1, -1)]
    if apply_ln:
        in_specs += [_full((1, H)), _full((1, H))]
        args += [ln_s.reshape(1, -1), ln_b.reshape(1, -1)]

    return pl.pallas_call(
        body,
        grid=(E2 // BE,),
        in_specs=in_specs,
        out_specs=_rows(BE, H),
        out_shape=jax.ShapeDtypeStruct((E2, H), F32),
    )(*args)


def _node_layer_mid(n, aggp, aggq, w1n, w1a, w2, b1, b2, ln_s, ln_b, wps, wpr):
    def body(n_ref, ap_ref, bp_ref, w1n_ref, w1a_ref, w2_ref, b1_ref, b2_ref,
             s_ref, lb_ref, wps_ref, wpr_ref, n2_ref, ps_ref, pr_ref):
        nn = n_ref[...]
        agg = (ap_ref[0] + ap_ref[1]) + (bp_ref[0] + bp_ref[1])
        h = jnp.maximum(
            _dot(nn, w1n_ref[...]) + _dot(agg, w1a_ref[...]) + b1_ref[...], 0.0)
        n2 = nn + _dot(h, w2_ref[...]) + b2_ref[...]
        n2 = _ln_rows(n2, s_ref[...], lb_ref[...])
        n2_ref[...] = n2
        ps_ref[...] = _dot(n2, wps_ref[...])
        pr_ref[...] = _dot(n2, wpr_ref[...])

    return pl.pallas_call(
        body,
        grid=(N // BN,),
        in_specs=[_rows(BN, H),
                  pl.BlockSpec((NC, BN, H), lambda i: (0, i, 0)),
                  pl.BlockSpec((NC, BN, H), lambda i: (0, i, 0)),
                  _full((H, H)), _full((H, H)), _full((H, H)),
                  _full((1, H)), _full((1, H)), _full((1, H)), _full((1, H)),
                  _full((H, H)), _full((H, H))],
        out_specs=[_rows(BN, H)] * 3,
        out_shape=[jax.ShapeDtypeStruct((N, H), F32)] * 3,
    )(n, aggp, aggq, w1n, w1a, w2, b1.reshape(1, -1), b2.reshape(1, -1),
      ln_s.reshape(1, -1), ln_b.reshape(1, -1), wps, wpr)


def _node_layer_last(n, aggp, aggq, w1n, w1a, w2, b1, b2, dw1, db1, dw2, db2):
    def body(n_ref, ap_ref, bp_ref, w1n_ref, w1a_ref, w2_ref, b1_ref, b2_ref,
             dw1_ref, db1_ref, dw2_ref, db2_ref, o_ref):
        nn = n_ref[...]
        agg = (ap_ref[0] + ap_ref[1]) + (bp_ref[0] + bp_ref[1])
        h = jnp.maximum(
            _dot(nn, w1n_ref[...]) + _dot(agg, w1a_ref[...]) + b1_ref[...], 0.0)
        n2 = nn + _dot(h, w2_ref[...]) + b2_ref[...]
        hd = jnp.maximum(_dot(n2, dw1_ref[...]) + db1_ref[...], 0.0)
        o_ref[...] = _dot(hd, dw2_ref[...]) + db2_ref[...]

    return pl.pallas_call(
        body,
        grid=(N // BN,),
        in_specs=[_rows(BN, H),
                  pl.BlockSpec((NC, BN, H), lambda i: (0, i, 0)),
                  pl.BlockSpec((NC, BN, H), lambda i: (0, i, 0)),
                  _full((H, H)), _full((H, H)), _full((H, H)),
                  _full((1, H)), _full((1, H)),
                  _full((H, H)), _full((1, H)), _full((H, 2 * H)), _full((1, 2 * H))],
        out_specs=_rows(BN, 2 * H),
        out_shape=jax.ShapeDtypeStruct((N, 2 * H), F32),
    )(n, aggp, aggq, w1n, w1a, w2, b1.reshape(1, -1), b2.reshape(1, -1),
      dw1, db1.reshape(1, -1), dw2, db2.reshape(1, -1))


def kernel(nodes, edges, senders, receivers, params):
    p = params
    pe_w1 = p['pe_W1']
    pn_w1 = p['pn_W1']

    sA3 = senders[:E2].reshape(NW, NCHUNK, GC)
    sB3 = senders[E2:].reshape(NW, NCHUNK, GC)
    rA3 = receivers[:E2].reshape(NW, NCHUNK, GC)
    rB3 = receivers[E2:].reshape(NW, NCHUNK, GC)

    eA = _edge_encoder(edges, p['enc_e_W1'], p['enc_e_b1'],
                       p['enc_e_W2'], p['enc_e_b2'], 0)
    eB = _edge_encoder(edges, p['enc_e_W1'], p['enc_e_b1'],
                       p['enc_e_W2'], p['enc_e_b2'], E2 // BE)
    n, ps, pr = _node_encoder(nodes, p['enc_n_W1'], p['enc_n_b1'],
                              p['enc_n_W2'], p['enc_n_b2'],
                              pe_w1[0, H:2 * H], pe_w1[0, 2 * H:])

    out = None
    for i in range(L):
        lnp = (p['ln_e_s'][i - 1] if i > 0 else None,
               p['ln_e_b'][i - 1] if i > 0 else None)
        gsA, grA = _sc_gather(ps, pr, sA3, rA3)
        eA = _edge_layer(eA, gsA, grA, pe_w1[i, :H], p['pe_W2'][i],
                         p['pe_b1'][i], p['pe_b2'][i], *lnp)
        gsB, grB = _sc_gather(ps, pr, sB3, rB3)
        aggpA = _sc_scatter(eA, rA3)
        eB = _edge_layer(eB, gsB, grB, pe_w1[i, :H], p['pe_W2'][i],
                         p['pe_b1'][i], p['pe_b2'][i], *lnp)
        aggpB = _sc_scatter(eB, rB3)
        if i < L - 1:
            n, ps, pr = _node_layer_mid(
                n, aggpA, aggpB, pn_w1[i, :H], pn_w1[i, H:], p['pn_W2'][i],
                p['pn_b1'][i], p['pn_b2'][i],
                p['ln_n_s'][i], p['ln_n_b'][i],
                pe_w1[i + 1, H:2 * H], pe_w1[i + 1, 2 * H:])
        else:
            out = _node_layer_last(
                n, aggpA, aggpB, pn_w1[i, :H], pn_w1[i, H:], p['pn_W2'][i],
                p['pn_b1'][i], p['pn_b2'][i],
                p['dec_W1'], p['dec_b1'], p['dec_W2'], p['dec_b2'])
    return out
